# baseline (speedup 1.0000x reference)
import jax
import jax.numpy as jnp
from jax.experimental import pallas as pl
from jax.experimental.pallas import tpu as pltpu

N = 10000
D = 128
BM = 80
S = 5


def _gcn_body(*refs):
    adj_refs = refs[:S]
    xfull_ref, w_ref, b_ref, xblk_ref, out_ref = refs[S:]
    xfull = xfull_ref[...]
    w = w_ref[...]
    b = b_ref[...]
    xblk = xblk_ref[...]
    for s in range(S):
        acc = jnp.dot(adj_refs[s][...], xfull,
                      preferred_element_type=jnp.float32)
        y = jnp.dot(acc, w, preferred_element_type=jnp.float32)
        out_ref[s * BM:(s + 1) * BM, :] = jnp.maximum(
            y + xblk[s * BM:(s + 1) * BM, :] + b, 0.0)


def _adj_spec(s):
    return pl.BlockSpec((BM, N), lambda i, s=s: (S * i + s, 0))


@jax.jit
def kernel(input, adj, W, b):
    x = input
    b2 = b.reshape(1, D)

    out = pl.pallas_call(
        _gcn_body,
        grid=(N // (S * BM),),
        in_specs=[_adj_spec(s) for s in range(S)] + [
            pl.BlockSpec((N, D), lambda i: (0, 0)),
            pl.BlockSpec((D, D), lambda i: (0, 0)),
            pl.BlockSpec((1, D), lambda i: (0, 0)),
            pl.BlockSpec((S * BM, D), lambda i: (i, 0)),
        ],
        out_specs=pl.BlockSpec((S * BM, D), lambda i: (i, 0)),
        out_shape=jax.ShapeDtypeStruct((N, D), jnp.float32),
        compiler_params=pltpu.CompilerParams(
            dimension_semantics=("arbitrary",),
        ),
    )(*([adj] * S), x, W, b2, x)

    return out
